# Initial kernel scaffold; baseline (speedup 1.0000x reference)
import jax
import jax.numpy as jnp
from jax.experimental import pallas as pl


def kernel(item_id, price, vocab, norm_mean, norm_var, bin_boundaries):
    raise NotImplementedError("write your pallas kernel here")


# speedup vs baseline: 33.9144x; 33.9144x over previous
import functools

import jax
import jax.numpy as jnp
from jax import lax
from jax.experimental import pallas as pl
from jax.experimental.pallas import tpu as pltpu
from jax.experimental.pallas import tpu_sc as plsc

_LANES = 16

_I32_SIGN_INT = -2147483648
_I32_MAG_INT = 0x7FFFFFFF


@functools.lru_cache(maxsize=None)
def _make_sc_lookup(vocab_n: int, batch_n: int):
    info = plsc.get_sparse_core_info()
    num_cores, num_subcores = info.num_cores, info.num_subcores
    num_workers = num_cores * num_subcores
    chunk = batch_n // num_workers
    assert chunk % _LANES == 0 and chunk * num_workers == batch_n
    steps = max(1, (vocab_n - 1).bit_length())
    mesh = plsc.VectorSubcoreMesh(core_axis_name="c", subcore_axis_name="s")

    @functools.partial(
        pl.kernel,
        out_type=jax.ShapeDtypeStruct((batch_n,), jnp.int32),
        mesh=mesh,
        scratch_types=[
            pltpu.VMEM((vocab_n,), jnp.int32),
            pltpu.VMEM((chunk,), jnp.int32),
            pltpu.VMEM((chunk,), jnp.int32),
        ],
        compiler_params=pltpu.CompilerParams(needs_layout_passes=False),
    )
    def lookup(vocab_hbm, ids_hbm, out_hbm, vocab_v, ids_v, res_v):
        wid = lax.axis_index("s") * num_cores + lax.axis_index("c")
        base = wid * chunk
        pltpu.sync_copy(vocab_hbm, vocab_v)
        pltpu.sync_copy(ids_hbm.at[pl.ds(base, chunk)], ids_v)

        def search_one(off):
            ids = ids_v[pl.ds(off, _LANES)]
            lo = jnp.zeros((_LANES,), jnp.int32)
            hi = jnp.full((_LANES,), vocab_n, jnp.int32)
            for _ in range(steps):
                mid = (lo + hi) >> 1
                v = plsc.load_gather(vocab_v, [jnp.minimum(mid, vocab_n - 1)])
                pred = v < ids
                lo = jnp.where(pred, mid + 1, lo)
                hi = jnp.where(pred, hi, mid)
            pos = jnp.minimum(lo, vocab_n - 1)
            vv = plsc.load_gather(vocab_v, [pos])
            res_v[pl.ds(off, _LANES)] = jnp.where(vv == ids, pos + 1, 0)

        def body(i, carry):
            search_one(i * (2 * _LANES))
            search_one(i * (2 * _LANES) + _LANES)
            return carry

        lax.fori_loop(0, chunk // (2 * _LANES), body, 0)
        pltpu.sync_copy(res_v, out_hbm.at[pl.ds(base, chunk)])

    return lookup


def _key_from_bits(b):
    return jnp.where(b < 0, b ^ jnp.int32(_I32_MAG_INT), b)


def _tc_stats_body(nbins, k1, k3, price_ref, bnd_ref, mv_ref,
                   clip_ref, disc_ref, norm_ref):
    p = price_ref[...]
    key = _key_from_bits(lax.bitcast_convert_type(p, jnp.int32))
    mn_key = jnp.min(key)

    a1 = jnp.int32(0)
    a3 = jnp.int32(0)
    for bit in range(31, -1, -1):
        mval = 1 << bit
        if mval >= 2**31:
            mval -= 2**32
        m = jnp.int32(mval)
        t1 = a1 | m
        t3 = a3 | m
        ts1 = t1 ^ jnp.int32(_I32_SIGN_INT)
        ts3 = t3 ^ jnp.int32(_I32_SIGN_INT)
        c = jnp.sum((key < ts1).astype(jnp.int32)
                    + ((key < ts3).astype(jnp.int32) << 16))
        c1 = c & jnp.int32(0xFFFF)
        c3 = c >> 16
        a1 = jnp.where(c1 <= k1, t1, a1)
        a3 = jnp.where(c3 <= k3, t3, a3)

    def key_to_f32(s):
        return lax.bitcast_convert_type(_key_from_bits(s), jnp.float32)

    q1 = key_to_f32(a1 ^ jnp.int32(_I32_SIGN_INT))
    q3 = key_to_f32(a3 ^ jnp.int32(_I32_SIGN_INT))
    mn = key_to_f32(mn_key)
    iqr = q3 - q1
    lower = jnp.maximum(q1 - 3.0 * iqr, mn)
    upper = q3 + 3.0 * iqr
    cp = jnp.clip(p, lower, upper)
    clip_ref[...] = cp
    norm_ref[...] = (cp - mv_ref[0]) / jnp.sqrt(mv_ref[1])

    acc = jnp.zeros(p.shape, jnp.int32)
    for j in range(nbins - 1):
        acc += (bnd_ref[j] <= cp).astype(jnp.int32)
    disc_ref[...] = acc


@functools.lru_cache(maxsize=None)
def _make_tc_stats(rows: int, cols: int, nbins: int):
    n = rows * cols
    k1 = (25 * (n - 1)) // 100
    k3 = (75 * (n - 1)) // 100
    return pl.pallas_call(
        functools.partial(_tc_stats_body, nbins, k1, k3),
        out_shape=(
            jax.ShapeDtypeStruct((rows, cols), jnp.float32),
            jax.ShapeDtypeStruct((rows, cols), jnp.int32),
            jax.ShapeDtypeStruct((rows, cols), jnp.float32),
        ),
        in_specs=[
            pl.BlockSpec(memory_space=pltpu.VMEM),
            pl.BlockSpec(memory_space=pltpu.SMEM),
            pl.BlockSpec(memory_space=pltpu.SMEM),
        ],
    )


def kernel(item_id, price, vocab, norm_mean, norm_var, bin_boundaries):
    batch_n = price.shape[0]
    vocab_n = vocab.shape[0]
    nbins = bin_boundaries.shape[0] + 1

    int_item_id = _make_sc_lookup(vocab_n, batch_n)(vocab, item_id)

    rows = batch_n // 128
    p2 = price.reshape(rows, 128)
    mv = jnp.stack([jnp.asarray(norm_mean, jnp.float32),
                    jnp.asarray(norm_var, jnp.float32)])
    clip2, disc2, norm2 = _make_tc_stats(rows, 128, nbins)(
        p2, bin_boundaries, mv)

    return (int_item_id,
            disc2.reshape(batch_n),
            norm2.reshape(batch_n),
            clip2.reshape(batch_n))


# speedup vs baseline: 36.5995x; 1.0792x over previous
import functools

import jax
import jax.numpy as jnp
from jax import lax
from jax.experimental import pallas as pl
from jax.experimental.pallas import tpu as pltpu
from jax.experimental.pallas import tpu_sc as plsc

_LANES = 16

_I32_SIGN_INT = -2147483648
_I32_MAG_INT = 0x7FFFFFFF


@functools.lru_cache(maxsize=None)
def _make_sc_lookup(vocab_n: int, batch_n: int):
    info = plsc.get_sparse_core_info()
    num_cores, num_subcores = info.num_cores, info.num_subcores
    num_workers = num_cores * num_subcores
    chunk = batch_n // num_workers
    assert chunk % _LANES == 0 and chunk * num_workers == batch_n
    steps = max(1, (vocab_n - 1).bit_length())
    mesh = plsc.VectorSubcoreMesh(core_axis_name="c", subcore_axis_name="s")

    @functools.partial(
        pl.kernel,
        out_type=jax.ShapeDtypeStruct((batch_n,), jnp.int32),
        mesh=mesh,
        scratch_types=[
            pltpu.VMEM((vocab_n + 8,), jnp.int32),
            pltpu.VMEM((chunk,), jnp.int32),
            pltpu.VMEM((chunk,), jnp.int32),
        ],
        compiler_params=pltpu.CompilerParams(needs_layout_passes=False),
    )
    def lookup(vocab_hbm, ids_hbm, out_hbm, vocab_v, ids_v, res_v):
        wid = lax.axis_index("s") * num_cores + lax.axis_index("c")
        base = wid * chunk
        pltpu.sync_copy(vocab_hbm, vocab_v.at[pl.ds(0, vocab_n)])
        pltpu.sync_copy(ids_hbm.at[pl.ds(base, chunk)], ids_v)

        interleave = 4

        def body(i, carry):
            offs = [i * (interleave * _LANES) + k * _LANES
                    for k in range(interleave)]
            ids = [ids_v[pl.ds(o, _LANES)] for o in offs]
            lo = [jnp.zeros((_LANES,), jnp.int32) for _ in offs]
            hi = [jnp.full((_LANES,), vocab_n, jnp.int32) for _ in offs]
            for _ in range(steps):
                mid = [(l + h) >> 1 for l, h in zip(lo, hi)]
                v = [plsc.load_gather(vocab_v, [m]) for m in mid]
                pred = [vk < idk for vk, idk in zip(v, ids)]
                lo = [jnp.where(p, m + 1, l)
                      for p, m, l in zip(pred, mid, lo)]
                hi = [jnp.where(p, h, m)
                      for p, m, h in zip(pred, mid, hi)]
            pos = [jnp.minimum(l, vocab_n - 1) for l in lo]
            vv = [plsc.load_gather(vocab_v, [p]) for p in pos]
            for o, vk, idk, p in zip(offs, vv, ids, pos):
                res_v[pl.ds(o, _LANES)] = jnp.where(vk == idk, p + 1, 0)
            return carry

        lax.fori_loop(0, chunk // (interleave * _LANES), body, 0)
        pltpu.sync_copy(res_v, out_hbm.at[pl.ds(base, chunk)])

    return lookup


def _key_from_bits(b):
    return jnp.where(b < 0, b ^ jnp.int32(_I32_MAG_INT), b)


def _tc_stats_body(nbins, k1, k3, price_ref, bnd_ref, mv_ref,
                   clip_ref, disc_ref, norm_ref):
    p = price_ref[...]
    key = _key_from_bits(lax.bitcast_convert_type(p, jnp.int32))
    mn_key = jnp.min(key)

    a1 = jnp.int32(0)
    a3 = jnp.int32(0)
    for bit in range(31, -1, -1):
        mval = 1 << bit
        if mval >= 2**31:
            mval -= 2**32
        m = jnp.int32(mval)
        t1 = a1 | m
        t3 = a3 | m
        ts1 = t1 ^ jnp.int32(_I32_SIGN_INT)
        ts3 = t3 ^ jnp.int32(_I32_SIGN_INT)
        c = jnp.sum((key < ts1).astype(jnp.int32)
                    + ((key < ts3).astype(jnp.int32) << 16))
        c1 = c & jnp.int32(0xFFFF)
        c3 = c >> 16
        a1 = jnp.where(c1 <= k1, t1, a1)
        a3 = jnp.where(c3 <= k3, t3, a3)

    def key_to_f32(s):
        return lax.bitcast_convert_type(_key_from_bits(s), jnp.float32)

    q1 = key_to_f32(a1 ^ jnp.int32(_I32_SIGN_INT))
    q3 = key_to_f32(a3 ^ jnp.int32(_I32_SIGN_INT))
    mn = key_to_f32(mn_key)
    iqr = q3 - q1
    lower = jnp.maximum(q1 - 3.0 * iqr, mn)
    upper = q3 + 3.0 * iqr
    cp = jnp.clip(p, lower, upper)
    clip_ref[...] = cp
    norm_ref[...] = (cp - mv_ref[0]) / jnp.sqrt(mv_ref[1])

    acc = jnp.zeros(p.shape, jnp.int32)
    for j in range(nbins - 1):
        acc += (bnd_ref[j] <= cp).astype(jnp.int32)
    disc_ref[...] = acc


@functools.lru_cache(maxsize=None)
def _make_tc_stats(rows: int, cols: int, nbins: int):
    n = rows * cols
    k1 = (25 * (n - 1)) // 100
    k3 = (75 * (n - 1)) // 100
    return pl.pallas_call(
        functools.partial(_tc_stats_body, nbins, k1, k3),
        out_shape=(
            jax.ShapeDtypeStruct((rows, cols), jnp.float32),
            jax.ShapeDtypeStruct((rows, cols), jnp.int32),
            jax.ShapeDtypeStruct((rows, cols), jnp.float32),
        ),
        in_specs=[
            pl.BlockSpec(memory_space=pltpu.VMEM),
            pl.BlockSpec(memory_space=pltpu.SMEM),
            pl.BlockSpec(memory_space=pltpu.SMEM),
        ],
    )


def kernel(item_id, price, vocab, norm_mean, norm_var, bin_boundaries):
    batch_n = price.shape[0]
    vocab_n = vocab.shape[0]
    nbins = bin_boundaries.shape[0] + 1

    int_item_id = _make_sc_lookup(vocab_n, batch_n)(vocab, item_id)

    rows = batch_n // 128
    p2 = price.reshape(rows, 128)
    mv = jnp.stack([jnp.asarray(norm_mean, jnp.float32),
                    jnp.asarray(norm_var, jnp.float32)])
    clip2, disc2, norm2 = _make_tc_stats(rows, 128, nbins)(
        p2, bin_boundaries, mv)

    return (int_item_id,
            disc2.reshape(batch_n),
            norm2.reshape(batch_n),
            clip2.reshape(batch_n))


# speedup vs baseline: 37.1619x; 1.0154x over previous
import functools

import jax
import jax.numpy as jnp
from jax import lax
from jax.experimental import pallas as pl
from jax.experimental.pallas import tpu as pltpu
from jax.experimental.pallas import tpu_sc as plsc

_LANES = 16

_I32_SIGN_INT = -2147483648
_I32_MAG_INT = 0x7FFFFFFF


@functools.lru_cache(maxsize=None)
def _make_sc_lookup(vocab_n: int, batch_n: int):
    info = plsc.get_sparse_core_info()
    num_cores, num_subcores = info.num_cores, info.num_subcores
    num_workers = num_cores * num_subcores
    chunk = batch_n // num_workers
    assert chunk % _LANES == 0 and chunk * num_workers == batch_n
    steps = max(1, (vocab_n - 1).bit_length())
    mesh = plsc.VectorSubcoreMesh(core_axis_name="c", subcore_axis_name="s")

    @functools.partial(
        pl.kernel,
        out_type=jax.ShapeDtypeStruct((batch_n,), jnp.int32),
        mesh=mesh,
        scratch_types=[
            pltpu.VMEM((vocab_n + 8,), jnp.int32),
            pltpu.VMEM((chunk,), jnp.int32),
            pltpu.VMEM((chunk,), jnp.int32),
        ],
        compiler_params=pltpu.CompilerParams(needs_layout_passes=False),
    )
    def lookup(vocab_hbm, ids_hbm, out_hbm, vocab_v, ids_v, res_v):
        wid = lax.axis_index("s") * num_cores + lax.axis_index("c")
        base = wid * chunk
        pltpu.sync_copy(vocab_hbm, vocab_v.at[pl.ds(0, vocab_n)])
        pltpu.sync_copy(ids_hbm.at[pl.ds(base, chunk)], ids_v)

        interleave = 8

        def body(i, carry):
            offs = [i * (interleave * _LANES) + k * _LANES
                    for k in range(interleave)]
            ids = [ids_v[pl.ds(o, _LANES)] for o in offs]
            lo = [jnp.zeros((_LANES,), jnp.int32) for _ in offs]
            hi = [jnp.full((_LANES,), vocab_n, jnp.int32) for _ in offs]
            for _ in range(steps):
                mid = [(l + h) >> 1 for l, h in zip(lo, hi)]
                v = [plsc.load_gather(vocab_v, [m]) for m in mid]
                pred = [vk < idk for vk, idk in zip(v, ids)]
                lo = [jnp.where(p, m + 1, l)
                      for p, m, l in zip(pred, mid, lo)]
                hi = [jnp.where(p, h, m)
                      for p, m, h in zip(pred, mid, hi)]
            pos = [jnp.minimum(l, vocab_n - 1) for l in lo]
            vv = [plsc.load_gather(vocab_v, [p]) for p in pos]
            for o, vk, idk, p in zip(offs, vv, ids, pos):
                res_v[pl.ds(o, _LANES)] = jnp.where(vk == idk, p + 1, 0)
            return carry

        lax.fori_loop(0, chunk // (interleave * _LANES), body, 0)
        pltpu.sync_copy(res_v, out_hbm.at[pl.ds(base, chunk)])

    return lookup


def _key_from_bits(b):
    return jnp.where(b < 0, b ^ jnp.int32(_I32_MAG_INT), b)


def _tc_stats_body(nbins, k1, k3, price_ref, bnd_ref, mv_ref,
                   clip_ref, disc_ref, norm_ref):
    p = price_ref[...]
    key = _key_from_bits(lax.bitcast_convert_type(p, jnp.int32))
    mn_key = jnp.min(key)

    a1 = jnp.int32(0)
    a3 = jnp.int32(0)
    for bit in range(31, -1, -1):
        mval = 1 << bit
        if mval >= 2**31:
            mval -= 2**32
        m = jnp.int32(mval)
        t1 = a1 | m
        t3 = a3 | m
        ts1 = t1 ^ jnp.int32(_I32_SIGN_INT)
        ts3 = t3 ^ jnp.int32(_I32_SIGN_INT)
        c = jnp.sum((key < ts1).astype(jnp.int32)
                    + ((key < ts3).astype(jnp.int32) << 16))
        c1 = c & jnp.int32(0xFFFF)
        c3 = c >> 16
        a1 = jnp.where(c1 <= k1, t1, a1)
        a3 = jnp.where(c3 <= k3, t3, a3)

    def key_to_f32(s):
        return lax.bitcast_convert_type(_key_from_bits(s), jnp.float32)

    q1 = key_to_f32(a1 ^ jnp.int32(_I32_SIGN_INT))
    q3 = key_to_f32(a3 ^ jnp.int32(_I32_SIGN_INT))
    mn = key_to_f32(mn_key)
    iqr = q3 - q1
    lower = jnp.maximum(q1 - 3.0 * iqr, mn)
    upper = q3 + 3.0 * iqr
    cp = jnp.clip(p, lower, upper)
    clip_ref[...] = cp
    norm_ref[...] = (cp - mv_ref[0]) / jnp.sqrt(mv_ref[1])

    acc = jnp.zeros(p.shape, jnp.int32)
    for j in range(nbins - 1):
        acc += (bnd_ref[j] <= cp).astype(jnp.int32)
    disc_ref[...] = acc


@functools.lru_cache(maxsize=None)
def _make_tc_stats(rows: int, cols: int, nbins: int):
    n = rows * cols
    k1 = (25 * (n - 1)) // 100
    k3 = (75 * (n - 1)) // 100
    return pl.pallas_call(
        functools.partial(_tc_stats_body, nbins, k1, k3),
        out_shape=(
            jax.ShapeDtypeStruct((rows, cols), jnp.float32),
            jax.ShapeDtypeStruct((rows, cols), jnp.int32),
            jax.ShapeDtypeStruct((rows, cols), jnp.float32),
        ),
        in_specs=[
            pl.BlockSpec(memory_space=pltpu.VMEM),
            pl.BlockSpec(memory_space=pltpu.SMEM),
            pl.BlockSpec(memory_space=pltpu.SMEM),
        ],
    )


def kernel(item_id, price, vocab, norm_mean, norm_var, bin_boundaries):
    batch_n = price.shape[0]
    vocab_n = vocab.shape[0]
    nbins = bin_boundaries.shape[0] + 1

    int_item_id = _make_sc_lookup(vocab_n, batch_n)(vocab, item_id)

    rows = batch_n // 128
    p2 = price.reshape(rows, 128)
    mv = jnp.stack([jnp.asarray(norm_mean, jnp.float32),
                    jnp.asarray(norm_var, jnp.float32)])
    clip2, disc2, norm2 = _make_tc_stats(rows, 128, nbins)(
        p2, bin_boundaries, mv)

    return (int_item_id,
            disc2.reshape(batch_n),
            norm2.reshape(batch_n),
            clip2.reshape(batch_n))


# speedup vs baseline: 41.2328x; 1.1095x over previous
import functools

import jax
import jax.numpy as jnp
from jax import lax
from jax.experimental import pallas as pl
from jax.experimental.pallas import tpu as pltpu
from jax.experimental.pallas import tpu_sc as plsc

_LANES = 16

_I32_SIGN_INT = -2147483648
_I32_MAG_INT = 0x7FFFFFFF


@functools.lru_cache(maxsize=None)
def _make_sc_lookup(vocab_n: int, batch_n: int):
    info = plsc.get_sparse_core_info()
    num_cores, num_subcores = info.num_cores, info.num_subcores
    num_workers = num_cores * num_subcores
    chunk = batch_n // num_workers
    assert chunk % (8 * _LANES) == 0 and chunk * num_workers == batch_n
    assert vocab_n % _LANES == 0
    samp_n = vocab_n // _LANES
    steps1 = max(1, (samp_n - 1).bit_length())
    steps2 = _LANES.bit_length()
    nblk = chunk // 128
    mesh = plsc.VectorSubcoreMesh(core_axis_name="c", subcore_axis_name="s")

    @functools.partial(
        pl.kernel,
        out_type=jax.ShapeDtypeStruct((batch_n,), jnp.int32),
        mesh=mesh,
        scratch_types=[
            pltpu.VMEM((samp_n + 8,), jnp.int32),
            pltpu.VMEM((chunk,), jnp.int32),
            pltpu.VMEM((nblk, 128), jnp.int32),
            pltpu.VMEM((chunk, _LANES), jnp.int32),
            pltpu.VMEM((chunk,), jnp.int32),
            pltpu.SemaphoreType.DMA,
        ],
        compiler_params=pltpu.CompilerParams(needs_layout_passes=False,
                                             use_tc_tiling_on_sc=False),
    )
    def lookup(samp_hbm, vocab2d_hbm, ids_hbm, out_hbm,
               samp_v, ids_v, rows_v, win_v, res_v, sem):
        wid = lax.axis_index("s") * num_cores + lax.axis_index("c")
        base = wid * chunk
        pltpu.sync_copy(samp_hbm, samp_v.at[pl.ds(0, samp_n)])
        pltpu.sync_copy(ids_hbm.at[pl.ds(base, chunk)], ids_v)

        def p1_body(i, carry):
            offs = [k * _LANES for k in range(8)]
            ids = [ids_v[pl.ds(i * 128 + o, _LANES)] for o in offs]
            lo = [jnp.zeros((_LANES,), jnp.int32) for _ in offs]
            hi = [jnp.full((_LANES,), samp_n, jnp.int32) for _ in offs]
            for _ in range(steps1):
                mid = [(l + h) >> 1 for l, h in zip(lo, hi)]
                v = [plsc.load_gather(samp_v, [m]) for m in mid]
                pred = [vk <= idk for vk, idk in zip(v, ids)]
                lo = [jnp.where(p, m + 1, l)
                      for p, m, l in zip(pred, mid, lo)]
                hi = [jnp.where(p, h, m)
                      for p, m, h in zip(pred, mid, hi)]
            for k, l in enumerate(lo):
                rows_v[i, pl.ds(k * _LANES, _LANES)] = jnp.clip(
                    l - 1, 0, samp_n - 1)
            return carry

        lax.fori_loop(0, nblk, p1_body, 0)

        copies = [
            pltpu.async_copy(vocab2d_hbm.at[rows_v.at[b]],
                             win_v.at[pl.ds(b * 128, 128)], sem)
            for b in range(nblk)
        ]

        for b in range(nblk):
            copies[b].wait()
            for j in range(8):
                off = b * 128 + j * _LANES
                ids = ids_v[pl.ds(off, _LANES)]
                r = rows_v[b, pl.ds(j * _LANES, _LANES)]
                idrow = lax.iota(jnp.int32, _LANES) + off
                lo = jnp.zeros((_LANES,), jnp.int32)
                hi = jnp.full((_LANES,), _LANES, jnp.int32)
                for _ in range(steps2):
                    mid = (lo + hi) >> 1
                    v = plsc.load_gather(
                        win_v, [idrow, jnp.minimum(mid, _LANES - 1)])
                    pred = v < ids
                    lo = jnp.where(pred, mid + 1, lo)
                    hi = jnp.where(pred, hi, mid)
                lo = jnp.minimum(lo, _LANES)
                pos = jnp.minimum(r * _LANES + lo, vocab_n - 1)
                v_in = plsc.load_gather(
                    win_v, [idrow, jnp.minimum(lo, _LANES - 1)])
                v_nxt = plsc.load_gather(
                    samp_v, [jnp.minimum(r + 1, samp_n - 1)])
                spill = (lo == _LANES) & (r < samp_n - 1)
                vv = jnp.where(spill, v_nxt, v_in)
                res_v[pl.ds(off, _LANES)] = jnp.where(vv == ids, pos + 1, 0)

        pltpu.sync_copy(res_v, out_hbm.at[pl.ds(base, chunk)])

    return lookup


def _key_from_bits(b):
    return jnp.where(b < 0, b ^ jnp.int32(_I32_MAG_INT), b)


def _tc_stats_body(nbins, k1, k3, price_ref, bnd_ref, mv_ref,
                   clip_ref, disc_ref, norm_ref):
    p = price_ref[...]
    key = _key_from_bits(lax.bitcast_convert_type(p, jnp.int32))
    mn_key = jnp.min(key)

    a1 = jnp.int32(0)
    a3 = jnp.int32(0)
    for bit in range(31, -1, -1):
        mval = 1 << bit
        if mval >= 2**31:
            mval -= 2**32
        m = jnp.int32(mval)
        t1 = a1 | m
        t3 = a3 | m
        ts1 = t1 ^ jnp.int32(_I32_SIGN_INT)
        ts3 = t3 ^ jnp.int32(_I32_SIGN_INT)
        c = jnp.sum((key < ts1).astype(jnp.int32)
                    + ((key < ts3).astype(jnp.int32) << 16))
        c1 = c & jnp.int32(0xFFFF)
        c3 = c >> 16
        a1 = jnp.where(c1 <= k1, t1, a1)
        a3 = jnp.where(c3 <= k3, t3, a3)

    def key_to_f32(s):
        return lax.bitcast_convert_type(_key_from_bits(s), jnp.float32)

    q1 = key_to_f32(a1 ^ jnp.int32(_I32_SIGN_INT))
    q3 = key_to_f32(a3 ^ jnp.int32(_I32_SIGN_INT))
    mn = key_to_f32(mn_key)
    iqr = q3 - q1
    lower = jnp.maximum(q1 - 3.0 * iqr, mn)
    upper = q3 + 3.0 * iqr
    cp = jnp.clip(p, lower, upper)
    clip_ref[...] = cp
    norm_ref[...] = (cp - mv_ref[0]) / jnp.sqrt(mv_ref[1])

    acc = jnp.zeros(p.shape, jnp.int32)
    for j in range(nbins - 1):
        acc += (bnd_ref[j] <= cp).astype(jnp.int32)
    disc_ref[...] = acc


@functools.lru_cache(maxsize=None)
def _make_tc_stats(rows: int, cols: int, nbins: int):
    n = rows * cols
    k1 = (25 * (n - 1)) // 100
    k3 = (75 * (n - 1)) // 100
    return pl.pallas_call(
        functools.partial(_tc_stats_body, nbins, k1, k3),
        out_shape=(
            jax.ShapeDtypeStruct((rows, cols), jnp.float32),
            jax.ShapeDtypeStruct((rows, cols), jnp.int32),
            jax.ShapeDtypeStruct((rows, cols), jnp.float32),
        ),
        in_specs=[
            pl.BlockSpec(memory_space=pltpu.VMEM),
            pl.BlockSpec(memory_space=pltpu.SMEM),
            pl.BlockSpec(memory_space=pltpu.SMEM),
        ],
    )


def kernel(item_id, price, vocab, norm_mean, norm_var, bin_boundaries):
    batch_n = price.shape[0]
    vocab_n = vocab.shape[0]
    nbins = bin_boundaries.shape[0] + 1

    samp = vocab[::_LANES]
    vocab2d = vocab.reshape(vocab_n // _LANES, _LANES)
    int_item_id = _make_sc_lookup(vocab_n, batch_n)(samp, vocab2d, item_id)

    rows = batch_n // 128
    p2 = price.reshape(rows, 128)
    mv = jnp.stack([jnp.asarray(norm_mean, jnp.float32),
                    jnp.asarray(norm_var, jnp.float32)])
    clip2, disc2, norm2 = _make_tc_stats(rows, 128, nbins)(
        p2, bin_boundaries, mv)

    return (int_item_id,
            disc2.reshape(batch_n),
            norm2.reshape(batch_n),
            clip2.reshape(batch_n))


# speedup vs baseline: 44.4037x; 1.0769x over previous
import functools

import jax
import jax.numpy as jnp
from jax import lax
from jax.experimental import pallas as pl
from jax.experimental.pallas import tpu as pltpu
from jax.experimental.pallas import tpu_sc as plsc

_LANES = 16

_I32_SIGN_INT = -2147483648
_I32_MAG_INT = 0x7FFFFFFF


@functools.lru_cache(maxsize=None)
def _make_sc_lookup(vocab_n: int, batch_n: int):
    info = plsc.get_sparse_core_info()
    num_cores, num_subcores = info.num_cores, info.num_subcores
    num_workers = num_cores * num_subcores
    chunk = batch_n // num_workers
    assert chunk % (8 * _LANES) == 0 and chunk * num_workers == batch_n
    assert vocab_n % _LANES == 0
    samp_n = vocab_n // _LANES
    steps1 = max(1, (samp_n - 1).bit_length())
    steps2 = _LANES.bit_length()
    nblk = chunk // 128
    mesh = plsc.VectorSubcoreMesh(core_axis_name="c", subcore_axis_name="s",
                                  num_cores=1)

    @functools.partial(
        pl.kernel,
        out_type=jax.ShapeDtypeStruct((batch_n,), jnp.int32),
        mesh=mesh,
        scratch_types=[
            pltpu.VMEM((samp_n + 8,), jnp.int32),
            pltpu.VMEM((chunk,), jnp.int32),
            pltpu.VMEM((nblk, 128), jnp.int32),
            pltpu.VMEM((chunk, _LANES), jnp.int32),
            pltpu.VMEM((chunk,), jnp.int32),
            pltpu.SemaphoreType.DMA,
        ],
        compiler_params=pltpu.CompilerParams(needs_layout_passes=False,
                                             use_tc_tiling_on_sc=False),
    )
    def lookup(samp_hbm, vocab2d_hbm, ids_hbm, out_hbm,
               samp_v, ids_v, rows_v, win_v, res_v, sem):
        wid = lax.axis_index("s") * num_cores + lax.axis_index("c")
        base = wid * chunk
        pltpu.sync_copy(samp_hbm, samp_v.at[pl.ds(0, samp_n)])
        pltpu.sync_copy(ids_hbm.at[pl.ds(base, chunk)], ids_v)

        def p1_body(i, carry):
            offs = [k * _LANES for k in range(8)]
            ids = [ids_v[pl.ds(i * 128 + o, _LANES)] for o in offs]
            lo = [jnp.zeros((_LANES,), jnp.int32) for _ in offs]
            hi = [jnp.full((_LANES,), samp_n, jnp.int32) for _ in offs]
            for _ in range(steps1):
                mid = [(l + h) >> 1 for l, h in zip(lo, hi)]
                v = [plsc.load_gather(samp_v, [m]) for m in mid]
                pred = [vk <= idk for vk, idk in zip(v, ids)]
                lo = [jnp.where(p, m + 1, l)
                      for p, m, l in zip(pred, mid, lo)]
                hi = [jnp.where(p, h, m)
                      for p, m, h in zip(pred, mid, hi)]
            for k, l in enumerate(lo):
                rows_v[i, pl.ds(k * _LANES, _LANES)] = jnp.clip(
                    l - 1, 0, samp_n - 1)
            return carry

        lax.fori_loop(0, nblk, p1_body, 0)

        copies = [
            pltpu.async_copy(vocab2d_hbm.at[rows_v.at[b]],
                             win_v.at[pl.ds(b * 128, 128)], sem)
            for b in range(nblk)
        ]

        for b in range(nblk):
            copies[b].wait()
            for j in range(8):
                off = b * 128 + j * _LANES
                ids = ids_v[pl.ds(off, _LANES)]
                r = rows_v[b, pl.ds(j * _LANES, _LANES)]
                idrow = lax.iota(jnp.int32, _LANES) + off
                lo = jnp.zeros((_LANES,), jnp.int32)
                hi = jnp.full((_LANES,), _LANES, jnp.int32)
                for _ in range(steps2):
                    mid = (lo + hi) >> 1
                    v = plsc.load_gather(
                        win_v, [idrow, jnp.minimum(mid, _LANES - 1)])
                    pred = v < ids
                    lo = jnp.where(pred, mid + 1, lo)
                    hi = jnp.where(pred, hi, mid)
                lo = jnp.minimum(lo, _LANES)
                pos = jnp.minimum(r * _LANES + lo, vocab_n - 1)
                v_in = plsc.load_gather(
                    win_v, [idrow, jnp.minimum(lo, _LANES - 1)])
                v_nxt = plsc.load_gather(
                    samp_v, [jnp.minimum(r + 1, samp_n - 1)])
                spill = (lo == _LANES) & (r < samp_n - 1)
                vv = jnp.where(spill, v_nxt, v_in)
                res_v[pl.ds(off, _LANES)] = jnp.where(vv == ids, pos + 1, 0)

        pltpu.sync_copy(res_v, out_hbm.at[pl.ds(base, chunk)])

    return lookup


def _key_from_bits(b):
    return jnp.where(b < 0, b ^ jnp.int32(_I32_MAG_INT), b)


def _tc_stats_body(nbins, k1, k3, price_ref, bnd_ref, mv_ref,
                   clip_ref, disc_ref, norm_ref):
    p = price_ref[...]
    key = _key_from_bits(lax.bitcast_convert_type(p, jnp.int32))
    mn_key = jnp.min(key)

    a1 = jnp.int32(0)
    a3 = jnp.int32(0)
    for bit in range(31, -1, -1):
        mval = 1 << bit
        if mval >= 2**31:
            mval -= 2**32
        m = jnp.int32(mval)
        t1 = a1 | m
        t3 = a3 | m
        ts1 = t1 ^ jnp.int32(_I32_SIGN_INT)
        ts3 = t3 ^ jnp.int32(_I32_SIGN_INT)
        c = jnp.sum((key < ts1).astype(jnp.int32)
                    + ((key < ts3).astype(jnp.int32) << 16))
        c1 = c & jnp.int32(0xFFFF)
        c3 = c >> 16
        a1 = jnp.where(c1 <= k1, t1, a1)
        a3 = jnp.where(c3 <= k3, t3, a3)

    def key_to_f32(s):
        return lax.bitcast_convert_type(_key_from_bits(s), jnp.float32)

    q1 = key_to_f32(a1 ^ jnp.int32(_I32_SIGN_INT))
    q3 = key_to_f32(a3 ^ jnp.int32(_I32_SIGN_INT))
    mn = key_to_f32(mn_key)
    iqr = q3 - q1
    lower = jnp.maximum(q1 - 3.0 * iqr, mn)
    upper = q3 + 3.0 * iqr
    cp = jnp.clip(p, lower, upper)
    clip_ref[...] = cp
    norm_ref[...] = (cp - mv_ref[0]) / jnp.sqrt(mv_ref[1])

    acc = jnp.zeros(p.shape, jnp.int32)
    for j in range(nbins - 1):
        acc += (bnd_ref[j] <= cp).astype(jnp.int32)
    disc_ref[...] = acc


@functools.lru_cache(maxsize=None)
def _make_tc_stats(rows: int, cols: int, nbins: int):
    n = rows * cols
    k1 = (25 * (n - 1)) // 100
    k3 = (75 * (n - 1)) // 100
    return pl.pallas_call(
        functools.partial(_tc_stats_body, nbins, k1, k3),
        out_shape=(
            jax.ShapeDtypeStruct((rows, cols), jnp.float32),
            jax.ShapeDtypeStruct((rows, cols), jnp.int32),
            jax.ShapeDtypeStruct((rows, cols), jnp.float32),
        ),
        in_specs=[
            pl.BlockSpec(memory_space=pltpu.VMEM),
            pl.BlockSpec(memory_space=pltpu.SMEM),
            pl.BlockSpec(memory_space=pltpu.SMEM),
        ],
    )


def kernel(item_id, price, vocab, norm_mean, norm_var, bin_boundaries):
    batch_n = price.shape[0]
    vocab_n = vocab.shape[0]
    nbins = bin_boundaries.shape[0] + 1

    samp = vocab[::_LANES]
    vocab2d = vocab.reshape(vocab_n // _LANES, _LANES)
    int_item_id = _make_sc_lookup(vocab_n, batch_n)(samp, vocab2d, item_id)

    rows = batch_n // 128
    p2 = price.reshape(rows, 128)
    mv = jnp.stack([jnp.asarray(norm_mean, jnp.float32),
                    jnp.asarray(norm_var, jnp.float32)])
    clip2, disc2, norm2 = _make_tc_stats(rows, 128, nbins)(
        p2, bin_boundaries, mv)

    return (int_item_id,
            disc2.reshape(batch_n),
            norm2.reshape(batch_n),
            clip2.reshape(batch_n))
